# Initial kernel scaffold; baseline (speedup 1.0000x reference)
import jax
import jax.numpy as jnp
from jax.experimental import pallas as pl


def kernel(x, edge_index, batch, weights):
    raise NotImplementedError("write your pallas kernel here")


# speedup vs baseline: 16.6311x; 16.6311x over previous
import functools

import jax
import jax.numpy as jnp
from jax import lax
from jax.experimental import pallas as pl
from jax.experimental.pallas import tpu as pltpu
from jax.experimental.pallas import tpu_sc as plsc

N = 10000
E = 320000
D = 128
DOUT = 64
NG = 64
DEPTH = 3
EPS = 1e-5

NP = 10240
NC = 2
NS = 16
NW = NC * NS
EW = E // NW
C = 80
K = EW // C
RW = N // NS
RB = 125
PW = NP // NS

_f32 = jnp.float32


def _sc_degree_body(dst2d, zeros_np, out, dst_all, ones_v, tmp_v, deg_sh):
    c = lax.axis_index("c")
    s = lax.axis_index("s")
    w = c * NS + s
    for i in range(C // 16):
        ones_v[pl.ds(i * 16, 16)] = jnp.ones((16,), _f32)
    pltpu.sync_copy(zeros_np.at[pl.ds(s * PW, PW)], tmp_v)
    pltpu.sync_copy(tmp_v, deg_sh.at[pl.ds(s * PW, PW)])
    plsc.subcore_barrier()
    pltpu.sync_copy(dst2d.at[pl.ds(w * K, K)], dst_all)

    def body(j, carry):
        pltpu.sync_copy(ones_v, deg_sh.at[dst_all.at[j]], add=True)
        return carry

    lax.fori_loop(0, K, body, 0)
    plsc.subcore_barrier()
    pltpu.sync_copy(deg_sh.at[pl.ds(s * PW, PW)], tmp_v)
    pltpu.sync_copy(tmp_v, out.at[c, pl.ds(s * PW, PW)])


def _sc_edges_body(g_hbm, src2d, dst2d, zeros_nd, out, src_all, dst_all,
                   rows_v, tmp_v, acc_sh, sem):
    c = lax.axis_index("c")
    s = lax.axis_index("s")
    w = c * NS + s

    def zbody(i, carry):
        pltpu.sync_copy(zeros_nd.at[pl.ds(s * RW + i * RB, RB)], tmp_v)
        pltpu.sync_copy(tmp_v, acc_sh.at[pl.ds(s * RW + i * RB, RB)])
        return carry

    lax.fori_loop(0, RW // RB, zbody, 0)
    plsc.subcore_barrier()
    pltpu.sync_copy(src2d.at[pl.ds(w * K, K)], src_all)
    pltpu.sync_copy(dst2d.at[pl.ds(w * K, K)], dst_all)

    def body(j, carry):
        pltpu.async_copy(g_hbm.at[src_all.at[j]], rows_v, sem).wait()
        pltpu.sync_copy(rows_v, acc_sh.at[dst_all.at[j]], add=True)
        return carry

    lax.fori_loop(0, K, body, 0)
    plsc.subcore_barrier()

    def obody(i, carry):
        pltpu.sync_copy(acc_sh.at[pl.ds(s * RW + i * RB, RB)], tmp_v)
        pltpu.sync_copy(tmp_v, out.at[c, pl.ds(s * RW + i * RB, RB)])
        return carry

    lax.fori_loop(0, RW // RB, obody, 0)


@functools.cache
def _get_sc_kernels():
    mesh = plsc.VectorSubcoreMesh(core_axis_name="c", subcore_axis_name="s",
                                  num_cores=NC, num_subcores=NS)
    params = pltpu.CompilerParams(use_tc_tiling_on_sc=False)
    sc_degree = pl.kernel(
        _sc_degree_body,
        out_type=jax.ShapeDtypeStruct((NC, NP), _f32),
        mesh=mesh,
        scratch_types=[
            pltpu.VMEM((K, C), jnp.int32),
            pltpu.VMEM((C,), _f32),
            pltpu.VMEM((PW,), _f32),
            pltpu.VMEM_SHARED((NP,), _f32),
        ],
        compiler_params=params,
    )
    sc_edges = pl.kernel(
        _sc_edges_body,
        out_type=jax.ShapeDtypeStruct((NC, N, D), _f32),
        mesh=mesh,
        scratch_types=[
            pltpu.VMEM((K, C), jnp.int32),
            pltpu.VMEM((K, C), jnp.int32),
            pltpu.VMEM((C, D), _f32),
            pltpu.VMEM((RB, D), _f32),
            pltpu.VMEM_SHARED((N, D), _f32),
            pltpu.SemaphoreType.DMA,
        ],
        compiler_params=params,
    )
    return sc_degree, sc_edges


def _dot(a, b):
    return jnp.dot(a, b, preferred_element_type=_f32)


def _pre_body(x, w1, b1, w2, b2, o):
    h = jnp.maximum(_dot(x[...], w1[...]) + b1[...], 0.0)
    o[...] = _dot(h, w2[...]) + b2[...]


_tc_pre = pl.pallas_call(
    _pre_body, out_shape=jax.ShapeDtypeStruct((N, D), _f32))


def _g0_body(h, wc, d1, d2, go, dvo):
    dinv = lax.rsqrt(d1[...] + d2[...] + 1.0)
    dvo[...] = dinv
    go[...] = _dot(h[...], wc[...]) * dinv


_tc_g0 = pl.pallas_call(
    _g0_body,
    out_shape=[jax.ShapeDtypeStruct((N, D), _f32),
               jax.ShapeDtypeStruct((N, 1), _f32)])


def _bn_relu(ss, g, dinv, bc, gamma, beta):
    u = dinv[...] * (ss[0] + ss[1] + g[...]) + bc[...]
    m = jnp.mean(u, axis=0, keepdims=True)
    v = jnp.mean(u * u, axis=0, keepdims=True) - m * m
    return jnp.maximum((u - m) * lax.rsqrt(v + EPS) * gamma[...] + beta[...],
                       0.0)


def _mid_body(ss, g, dinv, bc, gamma, beta, wn, go):
    hn = _bn_relu(ss, g, dinv, bc, gamma, beta)
    go[...] = _dot(hn, wn[...]) * dinv[...]


_tc_mid = pl.pallas_call(
    _mid_body, out_shape=jax.ShapeDtypeStruct((N, D), _f32))


def _fin_body(ss, g, dinv, bc, gamma, beta, v1, c1, v2, c2, b2d, yo):
    hn = _bn_relu(ss, g, dinv, bc, gamma, beta)
    p = jnp.maximum(_dot(hn, v1[...]) + c1[...], 0.0)
    p = _dot(p, v2[...]) + c2[...]
    oh = (b2d[...] == lax.broadcasted_iota(jnp.int32, (N, NG), 1))
    yo[...] = lax.dot_general(oh.astype(_f32), p, (((0,), (0,)), ((), ())),
                              preferred_element_type=_f32)


_tc_fin = pl.pallas_call(
    _fin_body, out_shape=jax.ShapeDtypeStruct((NG, DOUT), _f32))


def kernel(x, edge_index, batch, weights):
    w = list(weights)
    src2d = edge_index[0].reshape(E // C, C)
    dst2d = edge_index[1].reshape(E // C, C)
    zeros_np = jnp.zeros((NP,), _f32)
    zeros_nd = jnp.zeros((N, D), _f32)
    batch2d = batch.reshape(N, 1)

    sc_degree, sc_edges = _get_sc_kernels()
    degs = sc_degree(dst2d, zeros_np)
    d1 = degs[0, :N].reshape(N, 1)
    d2 = degs[1, :N].reshape(N, 1)

    h0 = _tc_pre(x, w[0], w[1].reshape(1, D), w[2], w[3].reshape(1, D))
    g, dinv = _tc_g0(h0, w[4], d1, d2)

    for l in range(DEPTH):
        bc = w[5 + 4 * l].reshape(1, D)
        gamma = w[6 + 4 * l].reshape(1, D)
        beta = w[7 + 4 * l].reshape(1, D)
        ss = sc_edges(g, src2d, dst2d, zeros_nd)
        if l < DEPTH - 1:
            g = _tc_mid(ss, g, dinv, bc, gamma, beta, w[8 + 4 * l])
        else:
            y = _tc_fin(ss, g, dinv, bc, gamma, beta,
                        w[16], w[17].reshape(1, DOUT),
                        w[18], w[19].reshape(1, DOUT), batch2d)
    return y


# speedup vs baseline: 26.6565x; 1.6028x over previous
import functools

import jax
import jax.numpy as jnp
from jax import lax
from jax.experimental import pallas as pl
from jax.experimental.pallas import tpu as pltpu
from jax.experimental.pallas import tpu_sc as plsc

N = 10000
E = 320000
D = 128
DOUT = 64
NG = 64
DEPTH = 3
EPS = 1e-5

NP = 10240
NC = 2
NS = 16
NW = NC * NS
EW = E // NW
C = 80
K = EW // C
RW = N // NS
NB = 5
RB = 125
DH = D // 2
ET = E // NS
KE = ET // C
PW = NP // NS

_f32 = jnp.float32


def _sc_degree_body(dst2d, zeros_np, out, dst_all, ones_v, tmp_v, deg_sh):
    c = lax.axis_index("c")
    s = lax.axis_index("s")
    w = c * NS + s
    for i in range(C // 16):
        ones_v[pl.ds(i * 16, 16)] = jnp.ones((16,), _f32)
    pltpu.sync_copy(zeros_np.at[pl.ds(s * PW, PW)], tmp_v)
    pltpu.sync_copy(tmp_v, deg_sh.at[pl.ds(s * PW, PW)])
    plsc.subcore_barrier()
    pltpu.sync_copy(dst2d.at[pl.ds(w * K, K)], dst_all)

    def body(j, carry):
        pltpu.sync_copy(ones_v, deg_sh.at[dst_all.at[j]], add=True)
        return carry

    lax.fori_loop(0, K, body, 0)
    plsc.subcore_barrier()
    pltpu.sync_copy(deg_sh.at[pl.ds(s * PW, PW)], tmp_v)
    pltpu.sync_copy(tmp_v, out.at[c, pl.ds(s * PW, PW)])


def _sc_edges_body(ghalves, src2d, dst2d, out, src_all, dst_all,
                   rows, tmp_v, acc_sh, sem_g, sem_s):
    c = lax.axis_index("c")
    s = lax.axis_index("s")
    table = ghalves.at[c]

    pltpu.sync_copy(src2d.at[pl.ds(s * KE, KE)], src_all)
    pltpu.sync_copy(dst2d.at[pl.ds(s * KE, KE)], dst_all)

    def zbody(i, carry):
        r0 = s * RW + i * RB
        pltpu.sync_copy(ghalves.at[c, pl.ds(r0, RB)], tmp_v)
        pltpu.sync_copy(tmp_v, acc_sh.at[pl.ds(r0, RB)])
        return carry

    lax.fori_loop(0, RW // RB, zbody, 0)
    plsc.subcore_barrier()

    for b in range(NB):
        pltpu.async_copy(table.at[src_all.at[b]], rows.at[b], sem_g.at[b])

    def body(gi, carry):
        for b in range(NB):
            j = gi * NB + b
            pltpu.make_async_copy(table.at[src_all.at[j]], rows.at[b],
                                  sem_g.at[b]).wait()
            pltpu.async_copy(rows.at[b], acc_sh.at[dst_all.at[j]], sem_s.at[b],
                             add=True)
        for b in range(NB):
            jn = gi * NB + b + NB
            pltpu.make_async_copy(rows.at[b], acc_sh.at[dst_all.at[0]],
                                  sem_s.at[b]).wait()

            @pl.when(jn < KE)
            def _():
                pltpu.async_copy(table.at[src_all.at[jn]], rows.at[b],
                                 sem_g.at[b])

        return carry

    lax.fori_loop(0, KE // NB, body, 0)
    plsc.subcore_barrier()

    def obody(i, carry):
        pltpu.sync_copy(acc_sh.at[pl.ds(s * RW + i * RB, RB)], tmp_v)
        pltpu.sync_copy(tmp_v, out.at[pl.ds(s * RW + i * RB, RB),
                                      pl.ds(c * DH, DH)])
        return carry

    lax.fori_loop(0, RW // RB, obody, 0)


@functools.cache
def _get_sc_kernels():
    mesh = plsc.VectorSubcoreMesh(core_axis_name="c", subcore_axis_name="s",
                                  num_cores=NC, num_subcores=NS)
    params = pltpu.CompilerParams(use_tc_tiling_on_sc=False)
    sc_degree = pl.kernel(
        _sc_degree_body,
        out_type=jax.ShapeDtypeStruct((NC, NP), _f32),
        mesh=mesh,
        scratch_types=[
            pltpu.VMEM((K, C), jnp.int32),
            pltpu.VMEM((C,), _f32),
            pltpu.VMEM((PW,), _f32),
            pltpu.VMEM_SHARED((NP,), _f32),
        ],
        compiler_params=params,
    )
    sc_edges = pl.kernel(
        _sc_edges_body,
        out_type=jax.ShapeDtypeStruct((N, D), _f32),
        mesh=mesh,
        scratch_types=[
            pltpu.VMEM((KE, C), jnp.int32),
            pltpu.VMEM((KE, C), jnp.int32),
            pltpu.VMEM((NB, C, DH), _f32),
            pltpu.VMEM((RB, DH), _f32),
            pltpu.VMEM_SHARED((N, DH), _f32),
            pltpu.SemaphoreType.DMA((NB,)),
            pltpu.SemaphoreType.DMA((NB,)),
        ],
        compiler_params=params,
    )
    return sc_degree, sc_edges


def _dot(a, b):
    return jnp.dot(a, b, preferred_element_type=_f32)


def _pre_body(x, w1, b1, w2, b2, wc, d1, d2, gho, dvo):
    h = jnp.maximum(_dot(x[...], w1[...]) + b1[...], 0.0)
    h = _dot(h, w2[...]) + b2[...]
    dinv = lax.rsqrt(d1[...] + d2[...] + 1.0)
    dvo[...] = dinv
    g = _dot(h, wc[...]) * dinv
    gho[0] = g[:, :DH]
    gho[1] = g[:, DH:]


_tc_pre = pl.pallas_call(
    _pre_body,
    out_shape=[jax.ShapeDtypeStruct((NC, N, DH), _f32),
               jax.ShapeDtypeStruct((N, 1), _f32)])


def _bn_relu(ss, dinv, bc, gamma, beta):
    u = dinv[...] * ss[...] + bc[...]
    m = jnp.mean(u, axis=0, keepdims=True)
    v = jnp.mean(u * u, axis=0, keepdims=True) - m * m
    return jnp.maximum((u - m) * lax.rsqrt(v + EPS) * gamma[...] + beta[...],
                       0.0)


def _mid_body(ss, dinv, bc, gamma, beta, wn, gho):
    hn = _bn_relu(ss, dinv, bc, gamma, beta)
    gn = _dot(hn, wn[...]) * dinv[...]
    gho[0] = gn[:, :DH]
    gho[1] = gn[:, DH:]


_tc_mid = pl.pallas_call(
    _mid_body, out_shape=jax.ShapeDtypeStruct((NC, N, DH), _f32))


def _fin_body(ss, dinv, bc, gamma, beta, v1, c1, v2, c2, b2d, yo):
    hn = _bn_relu(ss, dinv, bc, gamma, beta)
    p = jnp.maximum(_dot(hn, v1[...]) + c1[...], 0.0)
    p = _dot(p, v2[...]) + c2[...]
    oh = (b2d[...] == lax.broadcasted_iota(jnp.int32, (N, NG), 1))
    yo[...] = lax.dot_general(oh.astype(_f32), p, (((0,), (0,)), ((), ())),
                              preferred_element_type=_f32)


_tc_fin = pl.pallas_call(
    _fin_body, out_shape=jax.ShapeDtypeStruct((NG, DOUT), _f32))


def kernel(x, edge_index, batch, weights):
    w = list(weights)
    src2d = edge_index[0].reshape(E // C, C)
    dst2d = edge_index[1].reshape(E // C, C)
    zeros_np = jnp.zeros((NP,), _f32)
    batch2d = batch.reshape(N, 1)

    sc_degree, sc_edges = _get_sc_kernels()
    degs = sc_degree(dst2d, zeros_np)
    d1 = degs[0, :N].reshape(N, 1)
    d2 = degs[1, :N].reshape(N, 1)

    gh, dinv = _tc_pre(x, w[0], w[1].reshape(1, D), w[2], w[3].reshape(1, D),
                       w[4], d1, d2)

    for l in range(DEPTH):
        bc = w[5 + 4 * l].reshape(1, D)
        gamma = w[6 + 4 * l].reshape(1, D)
        beta = w[7 + 4 * l].reshape(1, D)
        ss = sc_edges(gh, src2d, dst2d)
        if l < DEPTH - 1:
            gh = _tc_mid(ss, dinv, bc, gamma, beta, w[8 + 4 * l])
        else:
            y = _tc_fin(ss, dinv, bc, gamma, beta,
                        w[16], w[17].reshape(1, DOUT),
                        w[18], w[19].reshape(1, DOUT), batch2d)
    return y


# speedup vs baseline: 27.7263x; 1.0401x over previous
import functools

import jax
import jax.numpy as jnp
from jax import lax
from jax.experimental import pallas as pl
from jax.experimental.pallas import tpu as pltpu
from jax.experimental.pallas import tpu_sc as plsc

N = 10000
E = 320000
D = 128
DOUT = 64
NG = 64
DEPTH = 3
EPS = 1e-5

NP = 10240
NC = 2
NS = 16
NW = NC * NS
EW = E // NW
C = 80
K = EW // C
RW = N // NS
NB = 5
RB = 125
DH = D // 2
ET = E // NS
KE = ET // C
PW = NP // NS

_f32 = jnp.float32


def _sc_degree_body(dst2d, zeros_np, out, dst_all, ones_v, tmp_v, deg_sh):
    c = lax.axis_index("c")
    s = lax.axis_index("s")
    w = c * NS + s
    for i in range(C // 16):
        ones_v[pl.ds(i * 16, 16)] = jnp.ones((16,), _f32)
    pltpu.sync_copy(zeros_np.at[pl.ds(s * PW, PW)], tmp_v)
    pltpu.sync_copy(tmp_v, deg_sh.at[pl.ds(s * PW, PW)])
    plsc.subcore_barrier()
    pltpu.sync_copy(dst2d.at[pl.ds(w * K, K)], dst_all)

    def body(j, carry):
        pltpu.sync_copy(ones_v, deg_sh.at[dst_all.at[j]], add=True)
        return carry

    lax.fori_loop(0, K, body, 0)
    plsc.subcore_barrier()
    pltpu.sync_copy(deg_sh.at[pl.ds(s * PW, PW)], tmp_v)
    pltpu.sync_copy(tmp_v, out.at[c, pl.ds(s * PW, PW)])


def _sc_edges_body(ghalves, src2d, dst2d, out, src_all, dst_all,
                   rows, tmp_v, acc_sh, sem_g, sem_s, sem_t, sem_u):
    c = lax.axis_index("c")
    s = lax.axis_index("s")
    table = ghalves.at[c]

    NI = RW // RB
    pltpu.async_copy(ghalves.at[c, pl.ds(s * RW, RB)], tmp_v.at[0], sem_t.at[0])
    pltpu.async_copy(src2d.at[pl.ds(s * KE, KE)], src_all, sem_g.at[0])
    pltpu.async_copy(dst2d.at[pl.ds(s * KE, KE)], dst_all, sem_g.at[1])
    for i in range(NI):
        b = i % 2
        r0 = s * RW + i * RB
        if i + 1 < NI:
            pltpu.async_copy(ghalves.at[c, pl.ds(r0 + RB, RB)],
                             tmp_v.at[1 - b], sem_t.at[1 - b])
        pltpu.make_async_copy(ghalves.at[c, pl.ds(r0, RB)], tmp_v.at[b],
                              sem_t.at[b]).wait()
        pltpu.async_copy(tmp_v.at[b], acc_sh.at[pl.ds(r0, RB)], sem_u.at[b])
    for b in range(2):
        pltpu.make_async_copy(tmp_v.at[b], acc_sh.at[pl.ds(s * RW, RB)],
                              sem_u.at[b]).wait()
    pltpu.make_async_copy(src2d.at[pl.ds(s * KE, KE)], src_all,
                          sem_g.at[0]).wait()
    pltpu.make_async_copy(dst2d.at[pl.ds(s * KE, KE)], dst_all,
                          sem_g.at[1]).wait()
    plsc.subcore_barrier()

    for b in range(NB):
        pltpu.async_copy(table.at[src_all.at[b]], rows.at[b], sem_g.at[b])

    def body(gi, carry):
        for b in range(NB):
            j = gi * NB + b
            pltpu.make_async_copy(table.at[src_all.at[j]], rows.at[b],
                                  sem_g.at[b]).wait()
            pltpu.async_copy(rows.at[b], acc_sh.at[dst_all.at[j]], sem_s.at[b],
                             add=True)
        for b in range(NB):
            jn = gi * NB + b + NB
            pltpu.make_async_copy(rows.at[b], acc_sh.at[dst_all.at[0]],
                                  sem_s.at[b]).wait()

            @pl.when(jn < KE)
            def _():
                pltpu.async_copy(table.at[src_all.at[jn]], rows.at[b],
                                 sem_g.at[b])

        return carry

    lax.fori_loop(0, KE // NB, body, 0)
    plsc.subcore_barrier()

    NI = RW // RB
    pltpu.async_copy(acc_sh.at[pl.ds(s * RW, RB)], tmp_v.at[0], sem_t.at[0])
    for i in range(NI):
        b = i % 2
        r0 = s * RW + i * RB
        pltpu.make_async_copy(acc_sh.at[pl.ds(r0, RB)], tmp_v.at[b],
                              sem_t.at[b]).wait()
        pltpu.async_copy(tmp_v.at[b], out.at[pl.ds(r0, RB), pl.ds(c * DH, DH)],
                         sem_u.at[b])
        if i + 1 < NI:
            if i + 2 < NI:
                pltpu.make_async_copy(
                    tmp_v.at[1 - b],
                    out.at[pl.ds(r0, RB), pl.ds(c * DH, DH)],
                    sem_u.at[1 - b]).wait()
            pltpu.async_copy(acc_sh.at[pl.ds(r0 + RB, RB)], tmp_v.at[1 - b],
                             sem_t.at[1 - b])
    for b in range(2):
        pltpu.make_async_copy(tmp_v.at[b],
                              out.at[pl.ds(s * RW, RB), pl.ds(c * DH, DH)],
                              sem_u.at[b]).wait()


@functools.cache
def _get_sc_kernels():
    mesh = plsc.VectorSubcoreMesh(core_axis_name="c", subcore_axis_name="s",
                                  num_cores=NC, num_subcores=NS)
    params = pltpu.CompilerParams(use_tc_tiling_on_sc=False)
    sc_degree = pl.kernel(
        _sc_degree_body,
        out_type=jax.ShapeDtypeStruct((NC, NP), _f32),
        mesh=mesh,
        scratch_types=[
            pltpu.VMEM((K, C), jnp.int32),
            pltpu.VMEM((C,), _f32),
            pltpu.VMEM((PW,), _f32),
            pltpu.VMEM_SHARED((NP,), _f32),
        ],
        compiler_params=params,
    )
    sc_edges = pl.kernel(
        _sc_edges_body,
        out_type=jax.ShapeDtypeStruct((N, D), _f32),
        mesh=mesh,
        scratch_types=[
            pltpu.VMEM((KE, C), jnp.int32),
            pltpu.VMEM((KE, C), jnp.int32),
            pltpu.VMEM((NB, C, DH), _f32),
            pltpu.VMEM((2, RB, DH), _f32),
            pltpu.VMEM_SHARED((N, DH), _f32),
            pltpu.SemaphoreType.DMA((NB,)),
            pltpu.SemaphoreType.DMA((NB,)),
            pltpu.SemaphoreType.DMA((2,)),
            pltpu.SemaphoreType.DMA((2,)),
        ],
        compiler_params=params,
    )
    return sc_degree, sc_edges


def _dot(a, b):
    return jnp.dot(a, b, preferred_element_type=_f32)


def _pre_body(x, w1, b1, w2, b2, wc, d1, d2, gho, dvo):
    h = jnp.maximum(_dot(x[...], w1[...]) + b1[...], 0.0)
    h = _dot(h, w2[...]) + b2[...]
    dinv = lax.rsqrt(d1[...] + d2[...] + 1.0)
    dvo[...] = dinv
    g = _dot(h, wc[...]) * dinv
    gho[0] = g[:, :DH]
    gho[1] = g[:, DH:]


_tc_pre = pl.pallas_call(
    _pre_body,
    out_shape=[jax.ShapeDtypeStruct((NC, N, DH), _f32),
               jax.ShapeDtypeStruct((N, 1), _f32)])


def _bn_relu(ss, dinv, bc, gamma, beta):
    u = dinv[...] * ss[...] + bc[...]
    m = jnp.mean(u, axis=0, keepdims=True)
    v = jnp.mean(u * u, axis=0, keepdims=True) - m * m
    return jnp.maximum((u - m) * lax.rsqrt(v + EPS) * gamma[...] + beta[...],
                       0.0)


def _mid_body(ss, dinv, bc, gamma, beta, wn, gho):
    hn = _bn_relu(ss, dinv, bc, gamma, beta)
    gn = _dot(hn, wn[...]) * dinv[...]
    gho[0] = gn[:, :DH]
    gho[1] = gn[:, DH:]


_tc_mid = pl.pallas_call(
    _mid_body, out_shape=jax.ShapeDtypeStruct((NC, N, DH), _f32))


def _fin_body(ss, dinv, bc, gamma, beta, v1, c1, v2, c2, b2d, yo):
    hn = _bn_relu(ss, dinv, bc, gamma, beta)
    p = jnp.maximum(_dot(hn, v1[...]) + c1[...], 0.0)
    p = _dot(p, v2[...]) + c2[...]
    oh = (b2d[...] == lax.broadcasted_iota(jnp.int32, (N, NG), 1))
    yo[...] = lax.dot_general(oh.astype(_f32), p, (((0,), (0,)), ((), ())),
                              preferred_element_type=_f32)


_tc_fin = pl.pallas_call(
    _fin_body, out_shape=jax.ShapeDtypeStruct((NG, DOUT), _f32))


def kernel(x, edge_index, batch, weights):
    w = list(weights)
    src2d = edge_index[0].reshape(E // C, C)
    dst2d = edge_index[1].reshape(E // C, C)
    zeros_np = jnp.zeros((NP,), _f32)
    batch2d = batch.reshape(N, 1)

    sc_degree, sc_edges = _get_sc_kernels()
    degs = sc_degree(dst2d, zeros_np)
    d1 = degs[0, :N].reshape(N, 1)
    d2 = degs[1, :N].reshape(N, 1)

    gh, dinv = _tc_pre(x, w[0], w[1].reshape(1, D), w[2], w[3].reshape(1, D),
                       w[4], d1, d2)

    for l in range(DEPTH):
        bc = w[5 + 4 * l].reshape(1, D)
        gamma = w[6 + 4 * l].reshape(1, D)
        beta = w[7 + 4 * l].reshape(1, D)
        ss = sc_edges(gh, src2d, dst2d)
        if l < DEPTH - 1:
            gh = _tc_mid(ss, dinv, bc, gamma, beta, w[8 + 4 * l])
        else:
            y = _tc_fin(ss, dinv, bc, gamma, beta,
                        w[16], w[17].reshape(1, DOUT),
                        w[18], w[19].reshape(1, DOUT), batch2d)
    return y


# speedup vs baseline: 28.0724x; 1.0125x over previous
import functools

import jax
import jax.numpy as jnp
from jax import lax
from jax.experimental import pallas as pl
from jax.experimental.pallas import tpu as pltpu
from jax.experimental.pallas import tpu_sc as plsc

N = 10000
E = 320000
D = 128
DOUT = 64
NG = 64
DEPTH = 3
EPS = 1e-5

NP = 10240
NC = 2
NS = 16
NW = NC * NS
EW = E // NW
C = 100
CD = 80
KD = EW // CD
RW = N // NS
NB = 5
RB = 125
DH = D // 2
ET = E // NS
KE = ET // C
PW = NP // NS

_f32 = jnp.float32


def _sc_degree_body(dst2d, zeros_np, out, dst_all, ones_v, tmp_v, deg_sh):
    c = lax.axis_index("c")
    s = lax.axis_index("s")
    w = c * NS + s
    for i in range(CD // 16):
        ones_v[pl.ds(i * 16, 16)] = jnp.ones((16,), _f32)
    pltpu.sync_copy(zeros_np.at[pl.ds(s * PW, PW)], tmp_v)
    pltpu.sync_copy(tmp_v, deg_sh.at[pl.ds(s * PW, PW)])
    plsc.subcore_barrier()
    pltpu.sync_copy(dst2d.at[pl.ds(w * KD, KD)], dst_all)

    def body(j, carry):
        pltpu.sync_copy(ones_v, deg_sh.at[dst_all.at[j]], add=True)
        return carry

    lax.fori_loop(0, KD, body, 0)
    plsc.subcore_barrier()
    pltpu.sync_copy(deg_sh.at[pl.ds(s * PW, PW)], tmp_v)
    pltpu.sync_copy(tmp_v, out.at[c, pl.ds(s * PW, PW)])


def _sc_edges_body(ghalves, src2d, dst2d, out, src_all, dst_all,
                   rows, tmp_v, acc_sh, sem_g, sem_s, sem_t, sem_u):
    c = lax.axis_index("c")
    s = lax.axis_index("s")
    table = ghalves.at[c]

    NI = RW // RB
    pltpu.async_copy(ghalves.at[c, pl.ds(s * RW, RB)], tmp_v.at[0], sem_t.at[0])
    pltpu.async_copy(src2d.at[pl.ds(s * KE, KE)], src_all, sem_g.at[0])
    pltpu.async_copy(dst2d.at[pl.ds(s * KE, KE)], dst_all, sem_g.at[1])
    for i in range(NI):
        b = i % 2
        r0 = s * RW + i * RB
        if i + 1 < NI:
            pltpu.async_copy(ghalves.at[c, pl.ds(r0 + RB, RB)],
                             tmp_v.at[1 - b], sem_t.at[1 - b])
        pltpu.make_async_copy(ghalves.at[c, pl.ds(r0, RB)], tmp_v.at[b],
                              sem_t.at[b]).wait()
        pltpu.async_copy(tmp_v.at[b], acc_sh.at[pl.ds(r0, RB)], sem_u.at[b])
    for b in range(2):
        pltpu.make_async_copy(tmp_v.at[b], acc_sh.at[pl.ds(s * RW, RB)],
                              sem_u.at[b]).wait()
    pltpu.make_async_copy(src2d.at[pl.ds(s * KE, KE)], src_all,
                          sem_g.at[0]).wait()
    pltpu.make_async_copy(dst2d.at[pl.ds(s * KE, KE)], dst_all,
                          sem_g.at[1]).wait()
    plsc.subcore_barrier()

    for b in range(NB):
        pltpu.async_copy(table.at[src_all.at[b]], rows.at[b], sem_g.at[b])

    def body(gi, carry):
        for b in range(NB):
            j = gi * NB + b
            pltpu.make_async_copy(table.at[src_all.at[j]], rows.at[b],
                                  sem_g.at[b]).wait()
            pltpu.async_copy(rows.at[b], acc_sh.at[dst_all.at[j]], sem_s.at[b],
                             add=True)
        for b in range(NB):
            jn = gi * NB + b + NB
            pltpu.make_async_copy(rows.at[b], acc_sh.at[dst_all.at[0]],
                                  sem_s.at[b]).wait()

            @pl.when(jn < KE)
            def _():
                pltpu.async_copy(table.at[src_all.at[jn]], rows.at[b],
                                 sem_g.at[b])

        return carry

    lax.fori_loop(0, KE // NB, body, 0)
    plsc.subcore_barrier()

    NI = RW // RB
    pltpu.async_copy(acc_sh.at[pl.ds(s * RW, RB)], tmp_v.at[0], sem_t.at[0])
    for i in range(NI):
        b = i % 2
        r0 = s * RW + i * RB
        pltpu.make_async_copy(acc_sh.at[pl.ds(r0, RB)], tmp_v.at[b],
                              sem_t.at[b]).wait()
        pltpu.async_copy(tmp_v.at[b], out.at[pl.ds(r0, RB), pl.ds(c * DH, DH)],
                         sem_u.at[b])
        if i + 1 < NI:
            if i + 2 < NI:
                pltpu.make_async_copy(
                    tmp_v.at[1 - b],
                    out.at[pl.ds(r0, RB), pl.ds(c * DH, DH)],
                    sem_u.at[1 - b]).wait()
            pltpu.async_copy(acc_sh.at[pl.ds(r0 + RB, RB)], tmp_v.at[1 - b],
                             sem_t.at[1 - b])
    for b in range(2):
        pltpu.make_async_copy(tmp_v.at[b],
                              out.at[pl.ds(s * RW, RB), pl.ds(c * DH, DH)],
                              sem_u.at[b]).wait()


@functools.cache
def _get_sc_kernels():
    mesh = plsc.VectorSubcoreMesh(core_axis_name="c", subcore_axis_name="s",
                                  num_cores=NC, num_subcores=NS)
    params = pltpu.CompilerParams(use_tc_tiling_on_sc=False)
    sc_degree = pl.kernel(
        _sc_degree_body,
        out_type=jax.ShapeDtypeStruct((NC, NP), _f32),
        mesh=mesh,
        scratch_types=[
            pltpu.VMEM((KD, CD), jnp.int32),
            pltpu.VMEM((CD,), _f32),
            pltpu.VMEM((PW,), _f32),
            pltpu.VMEM_SHARED((NP,), _f32),
        ],
        compiler_params=params,
    )
    sc_edges = pl.kernel(
        _sc_edges_body,
        out_type=jax.ShapeDtypeStruct((N, D), _f32),
        mesh=mesh,
        scratch_types=[
            pltpu.VMEM((KE, C), jnp.int32),
            pltpu.VMEM((KE, C), jnp.int32),
            pltpu.VMEM((NB, C, DH), _f32),
            pltpu.VMEM((2, RB, DH), _f32),
            pltpu.VMEM_SHARED((N, DH), _f32),
            pltpu.SemaphoreType.DMA((NB,)),
            pltpu.SemaphoreType.DMA((NB,)),
            pltpu.SemaphoreType.DMA((2,)),
            pltpu.SemaphoreType.DMA((2,)),
        ],
        compiler_params=params,
    )
    return sc_degree, sc_edges


def _dot(a, b):
    return jnp.dot(a, b, preferred_element_type=_f32)


def _pre_body(x, w1, b1, w2, b2, wc, d1, d2, gho, dvo):
    h = jnp.maximum(_dot(x[...], w1[...]) + b1[...], 0.0)
    h = _dot(h, w2[...]) + b2[...]
    dinv = lax.rsqrt(d1[...] + d2[...] + 1.0)
    dvo[...] = dinv
    g = _dot(h, wc[...]) * dinv
    gho[0] = g[:, :DH]
    gho[1] = g[:, DH:]


_tc_pre = pl.pallas_call(
    _pre_body,
    out_shape=[jax.ShapeDtypeStruct((NC, N, DH), _f32),
               jax.ShapeDtypeStruct((N, 1), _f32)])


def _bn_relu(ss, dinv, bc, gamma, beta):
    u = dinv[...] * ss[...] + bc[...]
    m = jnp.mean(u, axis=0, keepdims=True)
    v = jnp.mean(u * u, axis=0, keepdims=True) - m * m
    return jnp.maximum((u - m) * lax.rsqrt(v + EPS) * gamma[...] + beta[...],
                       0.0)


def _mid_body(ss, dinv, bc, gamma, beta, wn, gho):
    hn = _bn_relu(ss, dinv, bc, gamma, beta)
    gn = _dot(hn, wn[...]) * dinv[...]
    gho[0] = gn[:, :DH]
    gho[1] = gn[:, DH:]


_tc_mid = pl.pallas_call(
    _mid_body, out_shape=jax.ShapeDtypeStruct((NC, N, DH), _f32))


def _fin_body(ss, dinv, bc, gamma, beta, v1, c1, v2, c2, b2d, yo):
    hn = _bn_relu(ss, dinv, bc, gamma, beta)
    p = jnp.maximum(_dot(hn, v1[...]) + c1[...], 0.0)
    p = _dot(p, v2[...]) + c2[...]
    oh = (b2d[...] == lax.broadcasted_iota(jnp.int32, (N, NG), 1))
    yo[...] = lax.dot_general(oh.astype(_f32), p, (((0,), (0,)), ((), ())),
                              preferred_element_type=_f32)


_tc_fin = pl.pallas_call(
    _fin_body, out_shape=jax.ShapeDtypeStruct((NG, DOUT), _f32))


def kernel(x, edge_index, batch, weights):
    w = list(weights)
    src2d = edge_index[0].reshape(E // C, C)
    dst2d = edge_index[1].reshape(E // C, C)
    dst2d_deg = edge_index[1].reshape(E // CD, CD)
    zeros_np = jnp.zeros((NP,), _f32)
    batch2d = batch.reshape(N, 1)

    sc_degree, sc_edges = _get_sc_kernels()
    degs = sc_degree(dst2d_deg, zeros_np)
    d1 = degs[0, :N].reshape(N, 1)
    d2 = degs[1, :N].reshape(N, 1)

    gh, dinv = _tc_pre(x, w[0], w[1].reshape(1, D), w[2], w[3].reshape(1, D),
                       w[4], d1, d2)

    for l in range(DEPTH):
        bc = w[5 + 4 * l].reshape(1, D)
        gamma = w[6 + 4 * l].reshape(1, D)
        beta = w[7 + 4 * l].reshape(1, D)
        ss = sc_edges(gh, src2d, dst2d)
        if l < DEPTH - 1:
            gh = _tc_mid(ss, dinv, bc, gamma, beta, w[8 + 4 * l])
        else:
            y = _tc_fin(ss, dinv, bc, gamma, beta,
                        w[16], w[17].reshape(1, DOUT),
                        w[18], w[19].reshape(1, DOUT), batch2d)
    return y
